# baseline (speedup 1.0000x reference)
import functools

import jax
import jax.numpy as jnp
from jax import lax
from jax.experimental import pallas as pl
from jax.experimental.pallas import tpu as pltpu
from jax.experimental.pallas import tpu_sc as plsc

N = 10000
N_PAD = 10240
E = 160000
E_PAD = 163840
D = 256
DQ = 64
DH = 128
NC = 2
NS = 16
EPT = E_PAD // (NC * NS)
EPS = E_PAD // NS
ROWS_PER_SUB = N_PAD // NS
B = 128
CHUNK = 1024


def _deg_body(col_hbm, ew_hbm, mn_hbm, mf_hbm, zeros1_hbm, degp_hbm,
              colb, ewb, mnb, mfb, msgn, msgf, degn_sh, degf_sh):
    c = lax.axis_index("c")
    s = lax.axis_index("s")
    wid = s * NC + c
    ebase = wid * EPT

    pltpu.sync_copy(zeros1_hbm.at[pl.ds(s * ROWS_PER_SUB, ROWS_PER_SUB)],
                    degn_sh.at[pl.ds(s * ROWS_PER_SUB, ROWS_PER_SUB)])
    pltpu.sync_copy(zeros1_hbm.at[pl.ds(s * ROWS_PER_SUB, ROWS_PER_SUB)],
                    degf_sh.at[pl.ds(s * ROWS_PER_SUB, ROWS_PER_SUB)])
    plsc.subcore_barrier()

    def batch_body(j, _):
        off = ebase + j * B
        pltpu.sync_copy(col_hbm.at[pl.ds(off, B)], colb)
        pltpu.sync_copy(ew_hbm.at[pl.ds(off, B)], ewb)
        pltpu.sync_copy(mn_hbm.at[pl.ds(off, B)], mnb)
        pltpu.sync_copy(mf_hbm.at[pl.ds(off, B)], mfb)
        for g in range(B // 16):
            k = g * 16
            w = ewb[pl.ds(k, 16)]
            msgn[pl.ds(k, 16)] = w * mnb[pl.ds(k, 16)]
            msgf[pl.ds(k, 16)] = w * mfb[pl.ds(k, 16)]
        pltpu.sync_copy(msgn, degn_sh.at[colb], add=True)
        pltpu.sync_copy(msgf, degf_sh.at[colb], add=True)
        return 0

    lax.fori_loop(0, EPT // B, batch_body, 0)
    plsc.subcore_barrier()
    pltpu.sync_copy(degn_sh.at[pl.ds(s * ROWS_PER_SUB, ROWS_PER_SUB)],
                    degp_hbm.at[c, 0, pl.ds(s * ROWS_PER_SUB, ROWS_PER_SUB)])
    pltpu.sync_copy(degf_sh.at[pl.ds(s * ROWS_PER_SUB, ROWS_PER_SUB)],
                    degp_hbm.at[c, 1, pl.ds(s * ROWS_PER_SUB, ROWS_PER_SUB)])


def _sc_deg(col_p, ew_p, mn_p, mf_p, zeros1):
    mesh = plsc.VectorSubcoreMesh(core_axis_name="c", subcore_axis_name="s")
    f = pl.kernel(
        _deg_body,
        out_type=jax.ShapeDtypeStruct((NC, 2, N_PAD), jnp.float32),
        mesh=mesh,
        scratch_types=[
            pltpu.VMEM((B,), jnp.int32),
            pltpu.VMEM((B,), jnp.float32),
            pltpu.VMEM((B,), jnp.float32),
            pltpu.VMEM((B,), jnp.float32),
            pltpu.VMEM((B,), jnp.float32),
            pltpu.VMEM((B,), jnp.float32),
            pltpu.VMEM_SHARED((N_PAD,), jnp.float32),
            pltpu.VMEM_SHARED((N_PAD,), jnp.float32),
        ],
    )
    return f(col_p, ew_p, mn_p, mf_p, zeros1)


def _dinv_tc_body(degp_ref, dinv_ref):
    deg = degp_ref[0] + degp_ref[1] + 1.0
    dinv_ref[...] = lax.rsqrt(deg)


def _tc_dinv(degp):
    return pl.pallas_call(
        _dinv_tc_body,
        out_shape=jax.ShapeDtypeStruct((2, N_PAD), jnp.float32),
    )(degp)


def _gat_body(row_hbm, col_hbm, dn_hbm, df_hbm,
              dnr_hbm, dnc_hbm, dfr_hbm, dfc_hbm,
              rowb0, rowb1, colb0, colb1,
              ga0, ga1, ga2, ga3, gb0, gb1, gb2, gb3,
              sem_i, sem_g, sem_w):
    c = lax.axis_index("c")
    s = lax.axis_index("s")
    wid = s * NC + c
    ebase = wid * EPT
    NB = EPT // B
    rowbs = (rowb0, rowb1)
    colbs = (colb0, colb1)
    gsets = ((ga0, ga1, ga2, ga3), (gb0, gb1, gb2, gb3))
    outs = (dnr_hbm, dnc_hbm, dfr_hbm, dfc_hbm)

    def gather_all(bset):
        r, cc = rowbs[bset], colbs[bset]
        g = gsets[bset]
        pltpu.async_copy(dn_hbm.at[r], g[0], sem_g)
        pltpu.async_copy(dn_hbm.at[cc], g[1], sem_g)
        pltpu.async_copy(df_hbm.at[r], g[2], sem_g)
        pltpu.async_copy(df_hbm.at[cc], g[3], sem_g)

    def drain_gathers(bset):
        r, cc = rowbs[bset], colbs[bset]
        g = gsets[bset]
        pltpu.make_async_copy(dn_hbm.at[r], g[0], sem_g).wait()
        pltpu.make_async_copy(dn_hbm.at[cc], g[1], sem_g).wait()
        pltpu.make_async_copy(df_hbm.at[r], g[2], sem_g).wait()
        pltpu.make_async_copy(df_hbm.at[cc], g[3], sem_g).wait()

    def drain_writes(bset, off):
        for i in range(4):
            pltpu.make_async_copy(
                gsets[bset][i], outs[i].at[pl.ds(off, B)], sem_w).wait()

    pltpu.sync_copy(row_hbm.at[pl.ds(ebase, B)], rowb0)
    pltpu.sync_copy(col_hbm.at[pl.ds(ebase, B)], colb0)
    gather_all(0)

    def pair_body(jj, _):
        for par in range(2):
            j = jj * 2 + par
            off = ebase + j * B
            nxt = ebase + (j + 1) * B

            @pl.when(j + 1 < NB)
            def _():
                pltpu.async_copy(
                    row_hbm.at[pl.ds(nxt, B)], rowbs[1 - par], sem_i)
                pltpu.async_copy(
                    col_hbm.at[pl.ds(nxt, B)], colbs[1 - par], sem_i)

            drain_gathers(par)
            for i in range(4):
                pltpu.async_copy(
                    gsets[par][i], outs[i].at[pl.ds(off, B)], sem_w)

            @pl.when(j >= 1)
            def _():
                drain_writes(1 - par, off - B)

            @pl.when(j + 1 < NB)
            def _():
                pltpu.make_async_copy(
                    row_hbm.at[pl.ds(nxt, B)], rowbs[1 - par], sem_i).wait()
                pltpu.make_async_copy(
                    col_hbm.at[pl.ds(nxt, B)], colbs[1 - par], sem_i).wait()
                gather_all(1 - par)
        return 0

    lax.fori_loop(0, NB // 2, pair_body, 0)
    drain_writes(1, ebase + (NB - 1) * B)


def _sc_gather_dinv(row_p, col_p, dn, df):
    mesh = plsc.VectorSubcoreMesh(core_axis_name="c", subcore_axis_name="s")
    f = pl.kernel(
        _gat_body,
        out_type=tuple(
            jax.ShapeDtypeStruct((E_PAD,), jnp.float32) for _ in range(4)),
        mesh=mesh,
        scratch_types=(
            [pltpu.VMEM((B,), jnp.int32)] * 4
            + [pltpu.VMEM((B,), jnp.float32)] * 8
            + [pltpu.SemaphoreType.DMA] * 3
        ),
    )
    return f(row_p, col_p, dn, df)


def _norm_tc_body(dnr_ref, dnc_ref, dfr_ref, dfc_ref, ew_ref, mn_ref, mf_ref,
                  normcat_ref):
    w = ew_ref[...]
    normcat_ref[0] = dnr_ref[...] * dnc_ref[...] * (w * mn_ref[...])
    normcat_ref[1] = dfr_ref[...] * dfc_ref[...] * (w * mf_ref[...])


def _tc_norms(dnr, dnc, dfr, dfc, ew2, mn2, mf2):
    return pl.pallas_call(
        _norm_tc_body,
        out_shape=jax.ShapeDtypeStruct((2, E_PAD // 128, 128), jnp.float32),
    )(dnr, dnc, dfr, dfc, ew2, mn2, mf2)


def _norm16_tc_body(normn_ref, normf_ref, n16_ref):
    n16_ref[0] = jnp.broadcast_to(normn_ref[...], (normn_ref.shape[0], 16))
    n16_ref[1] = jnp.broadcast_to(normf_ref[...], (normf_ref.shape[0], 16))


def _tc_norm16(normn1, normf1):
    ECH = 2048
    grid = (E_PAD // ECH,)
    return pl.pallas_call(
        _norm16_tc_body,
        grid=grid,
        in_specs=[
            pl.BlockSpec((ECH, 1), lambda i: (i, 0)),
            pl.BlockSpec((ECH, 1), lambda i: (i, 0)),
        ],
        out_specs=pl.BlockSpec((2, ECH, 16), lambda i: (0, i, 0)),
        out_shape=jax.ShapeDtypeStruct((2, E_PAD, 16), jnp.float32),
    )(normn1, normf1)


RING = 2
B1 = 128


def _agg1_body(xh_hbm, row_hbm, col_hbm, n16_hbm, zeros_hbm,
               s_hbm,
               rowb0, rowb1, colb0, colb1, nr0, nr1, xr0, xr1, acc_sh,
               sem_g, sem_s):
    c = lax.axis_index("c")
    s = lax.axis_index("s")
    ebase = s * EPS
    NB = EPS // B1
    rowbs = (rowb0, rowb1)
    colbs = (colb0, colb1)
    nrs = (nr0, nr1)
    xrs = (xr0, xr1)

    for h in range(2):
        hoff = h * N_PAD

        def stage(bset, j):
            off = ebase + j * B1
            pltpu.sync_copy(col_hbm.at[pl.ds(off, B1)], colbs[bset])
            pltpu.sync_copy(row_hbm.at[pl.ds(off, B1)], rowbs[bset])
            if h:
                def ibody(i, _):
                    k = i * 16
                    rowbs[bset][pl.ds(k, 16)] = (
                        rowbs[bset][pl.ds(k, 16)] + hoff)
                    return 0
                lax.fori_loop(0, B1 // 16, ibody, 0)
            pltpu.async_copy(
                n16_hbm.at[c, pl.ds(off * 16, B1 * 16)], nrs[bset], sem_g)
            pltpu.async_copy(xh_hbm.at[rowbs[bset]], xrs[bset], sem_g)

        pltpu.sync_copy(zeros_hbm.at[pl.ds(s * ROWS_PER_SUB, ROWS_PER_SUB)],
                        acc_sh.at[pl.ds(s * ROWS_PER_SUB, ROWS_PER_SUB)])
        plsc.subcore_barrier()
        stage(0, 0)

        def pair_body(jj, _):
            for par in range(2):
                j2 = jj * 2 + par
                xr = xrs[par]
                nr = nrs[par]
                pltpu.make_async_copy(
                    n16_hbm.at[c, pl.ds((ebase + j2 * B1) * 16, B1 * 16)], nr,
                    sem_g).wait()
                pltpu.make_async_copy(
                    xh_hbm.at[rowbs[par]], xr, sem_g).wait()

                @pl.when(j2 >= 1)
                def _():
                    pltpu.make_async_copy(
                        xrs[1 - par], acc_sh.at[colbs[1 - par]], sem_s).wait()

                @pl.when(j2 + 1 < NB)
                def _():
                    stage(1 - par, j2 + 1)

                def scale_body(g, _):
                    for ke in range(16):
                        e = g * 16 + ke
                        nnv = nr[pl.ds(e * 16, 16)]
                        for k in range(DH // 16):
                            xr[e, pl.ds(k * 16, 16)] = (
                                xr[e, pl.ds(k * 16, 16)] * nnv)
                    return 0

                lax.fori_loop(0, B1 // 16, scale_body, 0)
                pltpu.async_copy(xr, acc_sh.at[colbs[par]], sem_s,
                                 add=True)
            return 0

        lax.fori_loop(0, NB // 2, pair_body, 0)
        pltpu.make_async_copy(xrs[1], acc_sh.at[colbs[1]], sem_s).wait()
        plsc.subcore_barrier()

        r0 = s * ROWS_PER_SUB
        pltpu.sync_copy(acc_sh.at[pl.ds(r0, ROWS_PER_SUB)],
                        s_hbm.at[c, h, pl.ds(r0, ROWS_PER_SUB)])
        plsc.subcore_barrier()


def _sc_agg1(xh, row_p, col_p, n16, zeros128):
    mesh = plsc.VectorSubcoreMesh(core_axis_name="c", subcore_axis_name="s")
    f = pl.kernel(
        _agg1_body,
        out_type=jax.ShapeDtypeStruct((2, 2, N_PAD, DH), jnp.float32),
        mesh=mesh,
        scratch_types=(
            [pltpu.VMEM((B1,), jnp.int32)] * 4
            + [pltpu.VMEM((B1 * 16,), jnp.float32)] * 2
            + [pltpu.VMEM((B1, DH), jnp.float32)] * 2
            + [pltpu.VMEM_SHARED((N_PAD, DH), jnp.float32)]
            + [pltpu.SemaphoreType.DMA, pltpu.SemaphoreType.DMA]
        ),
    )
    return f(xh, row_p, col_p, n16, zeros128)


def _dense1_body(x_ref, sn_ref, sf_ref, dn_ref, df_ref, orig_ref,
                 w1n_ref, w1f_ref, w24_ref, g1w_ref, g2w_ref,
                 b1n_ref, b1f_ref, g1b_ref, g2b_ref,
                 y4_ref, gsc_ref):
    dn2 = dn_ref[...] * dn_ref[...]
    df2 = df_ref[...] * df_ref[...]
    orig = orig_ref[...]

    br = x_ref.shape[1]
    accn = jnp.zeros((br, D), jnp.float32)
    accf = jnp.zeros((br, D), jnp.float32)
    g1a = jnp.zeros((br, 1), jnp.float32)
    for h in range(2):
        xh = x_ref[h]
        tnh = sn_ref[h] + dn2 * xh
        tfh = sf_ref[h] + df2 * xh
        w1n_h = w1n_ref[pl.ds(h * DH, DH), :]
        w1f_h = w1f_ref[pl.ds(h * DH, DH), :]
        accn = accn + jnp.dot(tnh, w1n_h, preferred_element_type=jnp.float32)
        accf = accf + jnp.dot(tfh, w1f_h, preferred_element_type=jnp.float32)
        g1a = g1a + jnp.sum(xh * g1w_ref[h:h + 1, :], axis=1, keepdims=True)

    hn1 = accn + b1n_ref[...]
    hf1 = accf + b1f_ref[...]
    g1 = jax.nn.sigmoid(g1a + g1b_ref[0, 0])
    h1 = jax.nn.relu((g1 * hn1 + (1.0 - g1) * hf1) * orig)
    y16 = jnp.dot(h1, w24_ref[...], preferred_element_type=jnp.float32)
    g2 = jax.nn.sigmoid(
        jnp.sum(h1 * g2w_ref[...], axis=1, keepdims=True) + g2b_ref[0, 0])
    y4_ref[...] = y16
    gsc_ref[...] = jnp.concatenate(
        [g2, dn2, df2, orig, jnp.zeros((br, 12), jnp.float32)], axis=1)


def _tc_dense1(xq, snq, sfq, dn2d, df2d, orig2d, w1n, w1f, w24, g1wq, g2wr,
               b1n, b1f, g1b, g2b):
    BR = 512
    grid = (N_PAD // BR,)
    return pl.pallas_call(
        _dense1_body,
        grid=grid,
        in_specs=[
            pl.BlockSpec((2, BR, DH), lambda i: (0, i, 0)),
            pl.BlockSpec((2, BR, DH), lambda i: (0, i, 0)),
            pl.BlockSpec((2, BR, DH), lambda i: (0, i, 0)),
            pl.BlockSpec((BR, 1), lambda i: (i, 0)),
            pl.BlockSpec((BR, 1), lambda i: (i, 0)),
            pl.BlockSpec((BR, 1), lambda i: (i, 0)),
            pl.BlockSpec((D, D), lambda i: (0, 0)),
            pl.BlockSpec((D, D), lambda i: (0, 0)),
            pl.BlockSpec((D, 16), lambda i: (0, 0)),
            pl.BlockSpec((2, DH), lambda i: (0, 0)),
            pl.BlockSpec((1, D), lambda i: (0, 0)),
            pl.BlockSpec((1, D), lambda i: (0, 0)),
            pl.BlockSpec((1, D), lambda i: (0, 0)),
            pl.BlockSpec((1, 1), lambda i: (0, 0)),
            pl.BlockSpec((1, 1), lambda i: (0, 0)),
        ],
        out_specs=[
            pl.BlockSpec((BR, 16), lambda i: (i, 0)),
            pl.BlockSpec((BR, 16), lambda i: (i, 0)),
        ],
        out_shape=(
            jax.ShapeDtypeStruct((N_PAD, 16), jnp.float32),
            jax.ShapeDtypeStruct((N_PAD, 16), jnp.float32),
        ),
    )(xq, snq, sfq, dn2d, df2d, orig2d, w1n, w1f, w24, g1wq, g2wr,
      b1n, b1f, g1b, g2b)


def _agg2_body(y0_hbm, y1_hbm, y2_hbm, y3_hbm, row_hbm, col_hbm,
               normn_hbm, normf_hbm, zeros1_hbm,
               u_hbm,
               normnv, normfv, rowb0, rowb1, colb0, colb1,
               ga0, ga1, ga2, ga3, gb0, gb1, gb2, gb3,
               ma0, ma1, ma2, ma3, mb0, mb1, mb2, mb3,
               a0, a1, a2, a3, sem_i, sem_g, sem_s):
    c = lax.axis_index("c")
    s = lax.axis_index("s")
    wid = s * NC + c
    ebase = wid * EPT
    ys = (y0_hbm, y1_hbm, y2_hbm, y3_hbm)
    rowbs = (rowb0, rowb1)
    colbs = (colb0, colb1)
    gs = ((ga0, ga1, ga2, ga3), (gb0, gb1, gb2, gb3))
    ms = ((ma0, ma1, ma2, ma3), (mb0, mb1, mb2, mb3))
    accs = (a0, a1, a2, a3)

    pltpu.sync_copy(normn_hbm.at[pl.ds(ebase, EPT)], normnv)
    pltpu.sync_copy(normf_hbm.at[pl.ds(ebase, EPT)], normfv)
    for acc in accs:
        pltpu.sync_copy(zeros1_hbm.at[pl.ds(s * ROWS_PER_SUB, ROWS_PER_SUB)],
                        acc.at[pl.ds(s * ROWS_PER_SUB, ROWS_PER_SUB)])
    plsc.subcore_barrier()
    NB = EPT // B

    def gather_all(par):
        for i in range(4):
            pltpu.async_copy(ys[i].at[rowbs[par]], gs[par][i], sem_g)

    def drain_gathers(par):
        for i in range(4):
            pltpu.make_async_copy(
                ys[i].at[rowbs[par]], gs[par][i], sem_g).wait()

    def drain_scatters(par):
        for i in range(4):
            pltpu.make_async_copy(
                ms[par][i], accs[i].at[colbs[par]], sem_s).wait()

    pltpu.sync_copy(row_hbm.at[pl.ds(ebase, B)], rowbs[0])
    pltpu.sync_copy(col_hbm.at[pl.ds(ebase, B)], colbs[0])
    gather_all(0)

    def pair_body(jj, _):
        for par in range(2):
            j = jj * 2 + par
            eb = j * B
            nxt = ebase + (j + 1) * B

            @pl.when(j >= 1)
            def _():
                drain_scatters(1 - par)

            @pl.when(j + 1 < NB)
            def _():
                pltpu.async_copy(
                    row_hbm.at[pl.ds(nxt, B)], rowbs[1 - par], sem_i)
                pltpu.async_copy(
                    col_hbm.at[pl.ds(nxt, B)], colbs[1 - par], sem_i)

            drain_gathers(par)

            def group_body(g, _):
                k = g * 16
                nng = normnv[pl.ds(eb + k, 16)]
                nfg = normfv[pl.ds(eb + k, 16)]
                ms[par][0][pl.ds(k, 16)] = gs[par][0][pl.ds(k, 16)] * nng
                ms[par][1][pl.ds(k, 16)] = gs[par][1][pl.ds(k, 16)] * nng
                ms[par][2][pl.ds(k, 16)] = gs[par][2][pl.ds(k, 16)] * nfg
                ms[par][3][pl.ds(k, 16)] = gs[par][3][pl.ds(k, 16)] * nfg
                return 0

            lax.fori_loop(0, B // 16, group_body, 0)
            for i in range(4):
                pltpu.async_copy(
                    ms[par][i], accs[i].at[colbs[par]], sem_s, add=True)

            @pl.when(j + 1 < NB)
            def _():
                pltpu.make_async_copy(
                    row_hbm.at[pl.ds(nxt, B)], rowbs[1 - par], sem_i).wait()
                pltpu.make_async_copy(
                    col_hbm.at[pl.ds(nxt, B)], colbs[1 - par], sem_i).wait()
                gather_all(1 - par)
        return 0

    lax.fori_loop(0, NB // 2, pair_body, 0)
    drain_scatters(1)
    plsc.subcore_barrier()

    r0 = s * ROWS_PER_SUB
    for i in range(4):
        pltpu.sync_copy(accs[i].at[pl.ds(r0, ROWS_PER_SUB)],
                        u_hbm.at[c, i, pl.ds(r0, ROWS_PER_SUB)])


def _sc_agg2(y0, y1, y2, y3, row_p, col_p, normn, normf, zeros1):
    mesh = plsc.VectorSubcoreMesh(core_axis_name="c", subcore_axis_name="s")
    f = pl.kernel(
        _agg2_body,
        out_type=jax.ShapeDtypeStruct((NC, 4, N_PAD), jnp.float32),
        mesh=mesh,
        scratch_types=(
            [pltpu.VMEM((EPT,), jnp.float32)] * 2
            + [pltpu.VMEM((B,), jnp.int32)] * 4
            + [pltpu.VMEM((B,), jnp.float32)] * 16
            + [pltpu.VMEM_SHARED((N_PAD,), jnp.float32)] * 4
            + [pltpu.SemaphoreType.DMA] * 3
        ),
    )
    return f(y0, y1, y2, y3, row_p, col_p, normn, normf, zeros1)


def _final_body(us_ref, y16_ref, gsc_ref, b2n_ref, b2f_ref, out_ref):
    g2 = gsc_ref[:, 0:1]
    dn2 = gsc_ref[:, 1:2]
    df2 = gsc_ref[:, 2:3]
    orig = gsc_ref[:, 3:4]
    un = us_ref[:, 0:2] + us_ref[:, 4:6]
    uf = us_ref[:, 2:4] + us_ref[:, 6:8]
    hn2 = un + dn2 * y16_ref[:, 0:2] + b2n_ref[...]
    hf2 = uf + df2 * y16_ref[:, 2:4] + b2f_ref[...]
    out_ref[...] = orig * (g2 * hn2 + (1.0 - g2) * hf2)


def _tc_final(us8, y16, gsc, b2n2d, b2f2d):
    return pl.pallas_call(
        _final_body,
        out_shape=jax.ShapeDtypeStruct((N_PAD, 2), jnp.float32),
    )(us8, y16, gsc, b2n2d, b2f2d)


def kernel(x, edge_index, edge_weight, node_node_mask, node_feat_mask,
           is_original_node, W1n, b1n, W1f, b1f, W2n, b2n, W2f, b2f,
           g1W, g1b, g2W, g2b):
    f32 = jnp.float32
    row = edge_index[0]
    col = edge_index[1]
    epad = E_PAD - E
    row_p = jnp.pad(row, (0, epad))
    col_p = jnp.pad(col, (0, epad))
    ew_p = jnp.pad(edge_weight, (0, epad))
    mn_p = jnp.pad(node_node_mask.astype(f32), (0, epad))
    mf_p = jnp.pad(node_feat_mask.astype(f32), (0, epad))

    npad = N_PAD - N
    x_pad = jnp.pad(x, ((0, npad), (0, 0)))
    xh = x_pad.reshape(N_PAD, 2, DH).transpose(1, 0, 2).reshape(2 * N_PAD, DH)
    xhb = (x_pad.reshape(N_PAD, 2, DH // 32, 2, 16)
           .transpose(1, 0, 2, 4, 3)
           .reshape(2 * N_PAD, DH).astype(jnp.bfloat16))
    orig2d = jnp.pad(is_original_node.astype(f32), (0, npad)).reshape(N_PAD, 1)

    zeros128 = jnp.zeros((N_PAD, DH), f32)
    zeros1 = jnp.zeros((N_PAD,), f32)

    degp = _sc_deg(col_p, ew_p, mn_p, mf_p, zeros1)
    dinv2 = _tc_dinv(degp)
    dn = jnp.asarray(dinv2[0])
    df = jnp.asarray(dinv2[1])

    dnr, dnc, dfr, dfc = _sc_gather_dinv(row_p, col_p, dn, df)
    e2 = (E_PAD // 128, 128)
    normcat = _tc_norms(
        dnr.reshape(e2), dnc.reshape(e2), dfr.reshape(e2), dfc.reshape(e2),
        ew_p.reshape(e2), mn_p.reshape(e2), mf_p.reshape(e2)
    ).reshape(2, E_PAD)
    normn = jnp.asarray(normcat[0])
    normf = jnp.asarray(normcat[1])
    n16 = _tc_norm16(normn.reshape(E_PAD, 1),
                     normf.reshape(E_PAD, 1)).reshape(2, E_PAD * 16)

    sboth = _sc_agg1(xh, row_p, col_p, n16, zeros128)
    snh = jnp.asarray(sboth[0])
    sfh = jnp.asarray(sboth[1])
    xh3 = xh.reshape(2, N_PAD, DH)

    w24 = jnp.concatenate(
        [W2n, W2f, jnp.zeros((D, 12), f32)], axis=1)
    g1wh = g1W[:, 0].reshape(2, DH)
    g2wr = g2W[:, 0].reshape(1, D)
    y16, gsc = _tc_dense1(
        xh3, snh, sfh, dn.reshape(N_PAD, 1), df.reshape(N_PAD, 1), orig2d,
        W1n, W1f, w24, g1wh, g2wr,
        b1n.reshape(1, D), b1f.reshape(1, D),
        g1b.reshape(1, 1), g2b.reshape(1, 1))

    ycols = [jnp.asarray(y16[:, i]) for i in range(4)]
    u = _sc_agg2(*ycols, row_p, col_p, normn, normf, zeros1)

    us8 = jnp.stack(
        [u[cc, i] for cc in range(2) for i in range(4)], axis=1)
    out = _tc_final(us8, y16, gsc, b2n.reshape(1, 2), b2f.reshape(1, 2))
    return out[:N]


# speedup vs baseline: 1.6641x; 1.6641x over previous
import functools

import jax
import jax.numpy as jnp
from jax import lax
from jax.experimental import pallas as pl
from jax.experimental.pallas import tpu as pltpu
from jax.experimental.pallas import tpu_sc as plsc

N = 10000
N_PAD = 10240
E = 160000
E_PAD = 163840
D = 256
DQ = 64
DH = 128
NC = 2
NS = 16
EPT = E_PAD // (NC * NS)
EPS = E_PAD // NS
ROWS_PER_SUB = N_PAD // NS
B = 128
CHUNK = 1024


def _deg_body(col_hbm, ew_hbm, mn_hbm, mf_hbm, zeros1_hbm, degp_hbm,
              colb0, colb1, ewb0, ewb1, mnb0, mnb1, mfb0, mfb1,
              msgn0, msgn1, msgf0, msgf1, degn_sh, degf_sh,
              sem_i, sem_s):
    c = lax.axis_index("c")
    s = lax.axis_index("s")
    wid = s * NC + c
    ebase = wid * EPT
    NB = EPT // B
    colbs = (colb0, colb1)
    ewbs = (ewb0, ewb1)
    mnbs = (mnb0, mnb1)
    mfbs = (mfb0, mfb1)
    msgns = (msgn0, msgn1)
    msgfs = (msgf0, msgf1)

    pltpu.sync_copy(zeros1_hbm.at[pl.ds(s * ROWS_PER_SUB, ROWS_PER_SUB)],
                    degn_sh.at[pl.ds(s * ROWS_PER_SUB, ROWS_PER_SUB)])
    pltpu.sync_copy(zeros1_hbm.at[pl.ds(s * ROWS_PER_SUB, ROWS_PER_SUB)],
                    degf_sh.at[pl.ds(s * ROWS_PER_SUB, ROWS_PER_SUB)])
    plsc.subcore_barrier()

    def stage(bset, off):
        pltpu.async_copy(col_hbm.at[pl.ds(off, B)], colbs[bset], sem_i)
        pltpu.async_copy(ew_hbm.at[pl.ds(off, B)], ewbs[bset], sem_i)
        pltpu.async_copy(mn_hbm.at[pl.ds(off, B)], mnbs[bset], sem_i)
        pltpu.async_copy(mf_hbm.at[pl.ds(off, B)], mfbs[bset], sem_i)

    def drain_stage(bset, off):
        pltpu.make_async_copy(
            col_hbm.at[pl.ds(off, B)], colbs[bset], sem_i).wait()
        pltpu.make_async_copy(
            ew_hbm.at[pl.ds(off, B)], ewbs[bset], sem_i).wait()
        pltpu.make_async_copy(
            mn_hbm.at[pl.ds(off, B)], mnbs[bset], sem_i).wait()
        pltpu.make_async_copy(
            mf_hbm.at[pl.ds(off, B)], mfbs[bset], sem_i).wait()

    def drain_scatters(bset):
        pltpu.make_async_copy(
            msgns[bset], degn_sh.at[colbs[bset]], sem_s).wait()
        pltpu.make_async_copy(
            msgfs[bset], degf_sh.at[colbs[bset]], sem_s).wait()

    stage(0, ebase)

    def pair_body(jj, _):
        for par in range(2):
            j = jj * 2 + par
            off = ebase + j * B

            @pl.when(j >= 1)
            def _():
                drain_scatters(1 - par)

            @pl.when(j + 1 < NB)
            def _():
                stage(1 - par, off + B)

            drain_stage(par, off)
            for g in range(B // 16):
                k = g * 16
                w = ewbs[par][pl.ds(k, 16)]
                msgns[par][pl.ds(k, 16)] = w * mnbs[par][pl.ds(k, 16)]
                msgfs[par][pl.ds(k, 16)] = w * mfbs[par][pl.ds(k, 16)]
            pltpu.async_copy(
                msgns[par], degn_sh.at[colbs[par]], sem_s, add=True)
            pltpu.async_copy(
                msgfs[par], degf_sh.at[colbs[par]], sem_s, add=True)
        return 0

    lax.fori_loop(0, NB // 2, pair_body, 0)
    drain_scatters(1)
    plsc.subcore_barrier()
    pltpu.sync_copy(degn_sh.at[pl.ds(s * ROWS_PER_SUB, ROWS_PER_SUB)],
                    degp_hbm.at[c, 0, pl.ds(s * ROWS_PER_SUB, ROWS_PER_SUB)])
    pltpu.sync_copy(degf_sh.at[pl.ds(s * ROWS_PER_SUB, ROWS_PER_SUB)],
                    degp_hbm.at[c, 1, pl.ds(s * ROWS_PER_SUB, ROWS_PER_SUB)])


def _sc_deg(col_p, ew_p, mn_p, mf_p, zeros1):
    mesh = plsc.VectorSubcoreMesh(core_axis_name="c", subcore_axis_name="s")
    f = pl.kernel(
        _deg_body,
        out_type=jax.ShapeDtypeStruct((NC, 2, N_PAD), jnp.float32),
        mesh=mesh,
        scratch_types=(
            [pltpu.VMEM((B,), jnp.int32)] * 2
            + [pltpu.VMEM((B,), jnp.float32)] * 10
            + [pltpu.VMEM_SHARED((N_PAD,), jnp.float32)] * 2
            + [pltpu.SemaphoreType.DMA] * 2
        ),
    )
    return f(col_p, ew_p, mn_p, mf_p, zeros1)


def _dinv_tc_body(degp_ref, dinv_ref):
    deg = degp_ref[0] + degp_ref[1] + 1.0
    dinv_ref[...] = lax.rsqrt(deg)


def _tc_dinv(degp):
    return pl.pallas_call(
        _dinv_tc_body,
        out_shape=jax.ShapeDtypeStruct((2, N_PAD), jnp.float32),
    )(degp)


def _gat_body(row_hbm, col_hbm, dn_hbm, df_hbm,
              dnr_hbm, dnc_hbm, dfr_hbm, dfc_hbm,
              rowb0, rowb1, colb0, colb1,
              ga0, ga1, ga2, ga3, gb0, gb1, gb2, gb3,
              sem_i, sem_g, sem_w):
    c = lax.axis_index("c")
    s = lax.axis_index("s")
    wid = s * NC + c
    ebase = wid * EPT
    NB = EPT // B
    rowbs = (rowb0, rowb1)
    colbs = (colb0, colb1)
    gsets = ((ga0, ga1, ga2, ga3), (gb0, gb1, gb2, gb3))
    outs = (dnr_hbm, dnc_hbm, dfr_hbm, dfc_hbm)

    def gather_all(bset):
        r, cc = rowbs[bset], colbs[bset]
        g = gsets[bset]
        pltpu.async_copy(dn_hbm.at[r], g[0], sem_g)
        pltpu.async_copy(dn_hbm.at[cc], g[1], sem_g)
        pltpu.async_copy(df_hbm.at[r], g[2], sem_g)
        pltpu.async_copy(df_hbm.at[cc], g[3], sem_g)

    def drain_gathers(bset):
        r, cc = rowbs[bset], colbs[bset]
        g = gsets[bset]
        pltpu.make_async_copy(dn_hbm.at[r], g[0], sem_g).wait()
        pltpu.make_async_copy(dn_hbm.at[cc], g[1], sem_g).wait()
        pltpu.make_async_copy(df_hbm.at[r], g[2], sem_g).wait()
        pltpu.make_async_copy(df_hbm.at[cc], g[3], sem_g).wait()

    def drain_writes(bset, off):
        for i in range(4):
            pltpu.make_async_copy(
                gsets[bset][i], outs[i].at[pl.ds(off, B)], sem_w).wait()

    pltpu.sync_copy(row_hbm.at[pl.ds(ebase, B)], rowb0)
    pltpu.sync_copy(col_hbm.at[pl.ds(ebase, B)], colb0)
    gather_all(0)

    def pair_body(jj, _):
        for par in range(2):
            j = jj * 2 + par
            off = ebase + j * B
            nxt = ebase + (j + 1) * B

            @pl.when(j + 1 < NB)
            def _():
                pltpu.async_copy(
                    row_hbm.at[pl.ds(nxt, B)], rowbs[1 - par], sem_i)
                pltpu.async_copy(
                    col_hbm.at[pl.ds(nxt, B)], colbs[1 - par], sem_i)

            drain_gathers(par)
            for i in range(4):
                pltpu.async_copy(
                    gsets[par][i], outs[i].at[pl.ds(off, B)], sem_w)

            @pl.when(j >= 1)
            def _():
                drain_writes(1 - par, off - B)

            @pl.when(j + 1 < NB)
            def _():
                pltpu.make_async_copy(
                    row_hbm.at[pl.ds(nxt, B)], rowbs[1 - par], sem_i).wait()
                pltpu.make_async_copy(
                    col_hbm.at[pl.ds(nxt, B)], colbs[1 - par], sem_i).wait()
                gather_all(1 - par)
        return 0

    lax.fori_loop(0, NB // 2, pair_body, 0)
    drain_writes(1, ebase + (NB - 1) * B)


def _sc_gather_dinv(row_p, col_p, dn, df):
    mesh = plsc.VectorSubcoreMesh(core_axis_name="c", subcore_axis_name="s")
    f = pl.kernel(
        _gat_body,
        out_type=tuple(
            jax.ShapeDtypeStruct((E_PAD,), jnp.float32) for _ in range(4)),
        mesh=mesh,
        scratch_types=(
            [pltpu.VMEM((B,), jnp.int32)] * 4
            + [pltpu.VMEM((B,), jnp.float32)] * 8
            + [pltpu.SemaphoreType.DMA] * 3
        ),
    )
    return f(row_p, col_p, dn, df)


def _norm_tc_body(dnr_ref, dnc_ref, dfr_ref, dfc_ref, ew_ref, mn_ref, mf_ref,
                  normcat_ref):
    w = ew_ref[...]
    normcat_ref[0] = dnr_ref[...] * dnc_ref[...] * (w * mn_ref[...])
    normcat_ref[1] = dfr_ref[...] * dfc_ref[...] * (w * mf_ref[...])


def _tc_norms(dnr, dnc, dfr, dfc, ew2, mn2, mf2):
    return pl.pallas_call(
        _norm_tc_body,
        out_shape=jax.ShapeDtypeStruct((2, E_PAD // 128, 128), jnp.float32),
    )(dnr, dnc, dfr, dfc, ew2, mn2, mf2)


RING = 2
B1 = 128


def _agg1_body(xh_hbm, row_hbm, col_hbm, normcat_hbm, zeros_hbm,
               s_hbm,
               normv, rowb0, rowb1, colb0, colb1, xr0, xr1, acc_sh,
               sem_g, sem_s):
    c = lax.axis_index("c")
    s = lax.axis_index("s")
    ebase = s * EPS
    NB = EPS // B1
    rowbs = (rowb0, rowb1)
    colbs = (colb0, colb1)
    xrs = (xr0, xr1)

    pltpu.sync_copy(normcat_hbm.at[c, pl.ds(ebase, EPS)], normv)

    for h in range(2):
        hoff = h * N_PAD

        def stage(bset, j):
            off = ebase + j * B1
            pltpu.sync_copy(col_hbm.at[pl.ds(off, B1)], colbs[bset])
            pltpu.sync_copy(row_hbm.at[pl.ds(off, B1)], rowbs[bset])
            if h:
                def ibody(i, _):
                    k = i * 16
                    rowbs[bset][pl.ds(k, 16)] = (
                        rowbs[bset][pl.ds(k, 16)] + hoff)
                    return 0
                lax.fori_loop(0, B1 // 16, ibody, 0)
            pltpu.async_copy(xh_hbm.at[rowbs[bset]], xrs[bset], sem_g)

        pltpu.sync_copy(zeros_hbm.at[pl.ds(s * ROWS_PER_SUB, ROWS_PER_SUB)],
                        acc_sh.at[pl.ds(s * ROWS_PER_SUB, ROWS_PER_SUB)])
        plsc.subcore_barrier()
        stage(0, 0)

        def pair_body(jj, _):
            for par in range(2):
                j2 = jj * 2 + par
                xr = xrs[par]
                pltpu.make_async_copy(
                    xh_hbm.at[rowbs[par]], xr, sem_g).wait()

                @pl.when(j2 >= 1)
                def _():
                    pltpu.make_async_copy(
                        xrs[1 - par], acc_sh.at[colbs[1 - par]], sem_s).wait()

                @pl.when(j2 + 1 < NB)
                def _():
                    stage(1 - par, j2 + 1)

                def scale_body(g, _):
                    nng = normv[pl.ds(j2 * B1 + g * 16, 16)]
                    for ke in range(16):
                        e = g * 16 + ke
                        nn = nng[ke]
                        for k in range(DH // 16):
                            xr[e, pl.ds(k * 16, 16)] = (
                                xr[e, pl.ds(k * 16, 16)] * nn)
                    return 0

                lax.fori_loop(0, B1 // 16, scale_body, 0)
                pltpu.async_copy(xr, acc_sh.at[colbs[par]], sem_s,
                                 add=True)
            return 0

        lax.fori_loop(0, NB // 2, pair_body, 0)
        pltpu.make_async_copy(xrs[1], acc_sh.at[colbs[1]], sem_s).wait()
        plsc.subcore_barrier()

        r0 = s * ROWS_PER_SUB
        pltpu.sync_copy(acc_sh.at[pl.ds(r0, ROWS_PER_SUB)],
                        s_hbm.at[c, h, pl.ds(r0, ROWS_PER_SUB)])
        plsc.subcore_barrier()


def _sc_agg1(xhb, row_p, col_p, normcat, zeros128):
    mesh = plsc.VectorSubcoreMesh(core_axis_name="c", subcore_axis_name="s")
    f = pl.kernel(
        _agg1_body,
        out_type=jax.ShapeDtypeStruct((2, 2, N_PAD, DH), jnp.float32),
        mesh=mesh,
        scratch_types=(
            [pltpu.VMEM((EPS,), jnp.float32)]
            + [pltpu.VMEM((B1,), jnp.int32)] * 4
            + [pltpu.VMEM((B1, DH), jnp.float32)] * 2
            + [pltpu.VMEM_SHARED((N_PAD, DH), jnp.float32)]
            + [pltpu.SemaphoreType.DMA, pltpu.SemaphoreType.DMA]
        ),
    )
    return f(xhb, row_p, col_p, normcat, zeros128)


def _dense1_body(x_ref, sn_ref, sf_ref, dn_ref, df_ref, orig_ref,
                 w1n_ref, w1f_ref, w24_ref, g1w_ref, g2w_ref,
                 b1n_ref, b1f_ref, g1b_ref, g2b_ref,
                 y4_ref, gsc_ref):
    dn2 = dn_ref[...] * dn_ref[...]
    df2 = df_ref[...] * df_ref[...]
    orig = orig_ref[...]

    br = x_ref.shape[1]
    accn = jnp.zeros((br, D), jnp.float32)
    accf = jnp.zeros((br, D), jnp.float32)
    g1a = jnp.zeros((br, 1), jnp.float32)
    for h in range(2):
        xh = x_ref[h]
        tnh = sn_ref[h] + dn2 * xh
        tfh = sf_ref[h] + df2 * xh
        w1n_h = w1n_ref[pl.ds(h * DH, DH), :]
        w1f_h = w1f_ref[pl.ds(h * DH, DH), :]
        accn = accn + jnp.dot(tnh, w1n_h, preferred_element_type=jnp.float32)
        accf = accf + jnp.dot(tfh, w1f_h, preferred_element_type=jnp.float32)
        g1a = g1a + jnp.sum(xh * g1w_ref[h:h + 1, :], axis=1, keepdims=True)

    hn1 = accn + b1n_ref[...]
    hf1 = accf + b1f_ref[...]
    g1 = jax.nn.sigmoid(g1a + g1b_ref[0, 0])
    h1 = jax.nn.relu((g1 * hn1 + (1.0 - g1) * hf1) * orig)
    y16 = jnp.dot(h1, w24_ref[...], preferred_element_type=jnp.float32)
    g2 = jax.nn.sigmoid(
        jnp.sum(h1 * g2w_ref[...], axis=1, keepdims=True) + g2b_ref[0, 0])
    y4_ref[...] = y16
    gsc_ref[...] = jnp.concatenate(
        [g2, dn2, df2, orig, jnp.zeros((br, 12), jnp.float32)], axis=1)


def _tc_dense1(xq, snq, sfq, dn2d, df2d, orig2d, w1n, w1f, w24, g1wq, g2wr,
               b1n, b1f, g1b, g2b):
    BR = 512
    grid = (N_PAD // BR,)
    return pl.pallas_call(
        _dense1_body,
        grid=grid,
        in_specs=[
            pl.BlockSpec((2, BR, DH), lambda i: (0, i, 0)),
            pl.BlockSpec((2, BR, DH), lambda i: (0, i, 0)),
            pl.BlockSpec((2, BR, DH), lambda i: (0, i, 0)),
            pl.BlockSpec((BR, 1), lambda i: (i, 0)),
            pl.BlockSpec((BR, 1), lambda i: (i, 0)),
            pl.BlockSpec((BR, 1), lambda i: (i, 0)),
            pl.BlockSpec((D, D), lambda i: (0, 0)),
            pl.BlockSpec((D, D), lambda i: (0, 0)),
            pl.BlockSpec((D, 16), lambda i: (0, 0)),
            pl.BlockSpec((2, DH), lambda i: (0, 0)),
            pl.BlockSpec((1, D), lambda i: (0, 0)),
            pl.BlockSpec((1, D), lambda i: (0, 0)),
            pl.BlockSpec((1, D), lambda i: (0, 0)),
            pl.BlockSpec((1, 1), lambda i: (0, 0)),
            pl.BlockSpec((1, 1), lambda i: (0, 0)),
        ],
        out_specs=[
            pl.BlockSpec((BR, 16), lambda i: (i, 0)),
            pl.BlockSpec((BR, 16), lambda i: (i, 0)),
        ],
        out_shape=(
            jax.ShapeDtypeStruct((N_PAD, 16), jnp.float32),
            jax.ShapeDtypeStruct((N_PAD, 16), jnp.float32),
        ),
    )(xq, snq, sfq, dn2d, df2d, orig2d, w1n, w1f, w24, g1wq, g2wr,
      b1n, b1f, g1b, g2b)


def _agg2_body(y0_hbm, y1_hbm, y2_hbm, y3_hbm, row_hbm, col_hbm,
               normn_hbm, normf_hbm, zeros1_hbm,
               u_hbm,
               normnv, normfv, rowb0, rowb1, colb0, colb1,
               ga0, ga1, ga2, ga3, gb0, gb1, gb2, gb3,
               ma0, ma1, ma2, ma3, mb0, mb1, mb2, mb3,
               a0, a1, a2, a3, sem_i, sem_g, sem_s):
    c = lax.axis_index("c")
    s = lax.axis_index("s")
    wid = s * NC + c
    ebase = wid * EPT
    ys = (y0_hbm, y1_hbm, y2_hbm, y3_hbm)
    rowbs = (rowb0, rowb1)
    colbs = (colb0, colb1)
    gs = ((ga0, ga1, ga2, ga3), (gb0, gb1, gb2, gb3))
    ms = ((ma0, ma1, ma2, ma3), (mb0, mb1, mb2, mb3))
    accs = (a0, a1, a2, a3)

    pltpu.sync_copy(normn_hbm.at[pl.ds(ebase, EPT)], normnv)
    pltpu.sync_copy(normf_hbm.at[pl.ds(ebase, EPT)], normfv)
    for acc in accs:
        pltpu.sync_copy(zeros1_hbm.at[pl.ds(s * ROWS_PER_SUB, ROWS_PER_SUB)],
                        acc.at[pl.ds(s * ROWS_PER_SUB, ROWS_PER_SUB)])
    plsc.subcore_barrier()
    NB = EPT // B

    def gather_all(par):
        for i in range(4):
            pltpu.async_copy(ys[i].at[rowbs[par]], gs[par][i], sem_g)

    def drain_gathers(par):
        for i in range(4):
            pltpu.make_async_copy(
                ys[i].at[rowbs[par]], gs[par][i], sem_g).wait()

    def drain_scatters(par):
        for i in range(4):
            pltpu.make_async_copy(
                ms[par][i], accs[i].at[colbs[par]], sem_s).wait()

    pltpu.sync_copy(row_hbm.at[pl.ds(ebase, B)], rowbs[0])
    pltpu.sync_copy(col_hbm.at[pl.ds(ebase, B)], colbs[0])
    gather_all(0)

    def pair_body(jj, _):
        for par in range(2):
            j = jj * 2 + par
            eb = j * B
            nxt = ebase + (j + 1) * B

            @pl.when(j >= 1)
            def _():
                drain_scatters(1 - par)

            @pl.when(j + 1 < NB)
            def _():
                pltpu.async_copy(
                    row_hbm.at[pl.ds(nxt, B)], rowbs[1 - par], sem_i)
                pltpu.async_copy(
                    col_hbm.at[pl.ds(nxt, B)], colbs[1 - par], sem_i)

            drain_gathers(par)

            def group_body(g, _):
                k = g * 16
                nng = normnv[pl.ds(eb + k, 16)]
                nfg = normfv[pl.ds(eb + k, 16)]
                ms[par][0][pl.ds(k, 16)] = gs[par][0][pl.ds(k, 16)] * nng
                ms[par][1][pl.ds(k, 16)] = gs[par][1][pl.ds(k, 16)] * nng
                ms[par][2][pl.ds(k, 16)] = gs[par][2][pl.ds(k, 16)] * nfg
                ms[par][3][pl.ds(k, 16)] = gs[par][3][pl.ds(k, 16)] * nfg
                return 0

            lax.fori_loop(0, B // 16, group_body, 0)
            for i in range(4):
                pltpu.async_copy(
                    ms[par][i], accs[i].at[colbs[par]], sem_s, add=True)

            @pl.when(j + 1 < NB)
            def _():
                pltpu.make_async_copy(
                    row_hbm.at[pl.ds(nxt, B)], rowbs[1 - par], sem_i).wait()
                pltpu.make_async_copy(
                    col_hbm.at[pl.ds(nxt, B)], colbs[1 - par], sem_i).wait()
                gather_all(1 - par)
        return 0

    lax.fori_loop(0, NB // 2, pair_body, 0)
    drain_scatters(1)
    plsc.subcore_barrier()

    r0 = s * ROWS_PER_SUB
    for i in range(4):
        pltpu.sync_copy(accs[i].at[pl.ds(r0, ROWS_PER_SUB)],
                        u_hbm.at[c, i, pl.ds(r0, ROWS_PER_SUB)])


def _sc_agg2(y0, y1, y2, y3, row_p, col_p, normn, normf, zeros1):
    mesh = plsc.VectorSubcoreMesh(core_axis_name="c", subcore_axis_name="s")
    f = pl.kernel(
        _agg2_body,
        out_type=jax.ShapeDtypeStruct((NC, 4, N_PAD), jnp.float32),
        mesh=mesh,
        scratch_types=(
            [pltpu.VMEM((EPT,), jnp.float32)] * 2
            + [pltpu.VMEM((B,), jnp.int32)] * 4
            + [pltpu.VMEM((B,), jnp.float32)] * 16
            + [pltpu.VMEM_SHARED((N_PAD,), jnp.float32)] * 4
            + [pltpu.SemaphoreType.DMA] * 3
        ),
    )
    return f(y0, y1, y2, y3, row_p, col_p, normn, normf, zeros1)


def _final_body(us_ref, y16_ref, gsc_ref, b2n_ref, b2f_ref, out_ref):
    g2 = gsc_ref[:, 0:1]
    dn2 = gsc_ref[:, 1:2]
    df2 = gsc_ref[:, 2:3]
    orig = gsc_ref[:, 3:4]
    un = us_ref[:, 0:2] + us_ref[:, 4:6]
    uf = us_ref[:, 2:4] + us_ref[:, 6:8]
    hn2 = un + dn2 * y16_ref[:, 0:2] + b2n_ref[...]
    hf2 = uf + df2 * y16_ref[:, 2:4] + b2f_ref[...]
    out_ref[...] = orig * (g2 * hn2 + (1.0 - g2) * hf2)


def _tc_final(us8, y16, gsc, b2n2d, b2f2d):
    return pl.pallas_call(
        _final_body,
        out_shape=jax.ShapeDtypeStruct((N_PAD, 2), jnp.float32),
    )(us8, y16, gsc, b2n2d, b2f2d)


def kernel(x, edge_index, edge_weight, node_node_mask, node_feat_mask,
           is_original_node, W1n, b1n, W1f, b1f, W2n, b2n, W2f, b2f,
           g1W, g1b, g2W, g2b):
    f32 = jnp.float32
    row = edge_index[0]
    col = edge_index[1]
    epad = E_PAD - E
    row_p = jnp.pad(row, (0, epad))
    col_p = jnp.pad(col, (0, epad))
    ew_p = jnp.pad(edge_weight, (0, epad))
    mn_p = jnp.pad(node_node_mask.astype(f32), (0, epad))
    mf_p = jnp.pad(node_feat_mask.astype(f32), (0, epad))

    npad = N_PAD - N
    x_pad = jnp.pad(x, ((0, npad), (0, 0)))
    xh = x_pad.reshape(N_PAD, 2, DH).transpose(1, 0, 2).reshape(2 * N_PAD, DH)
    xhb = (x_pad.reshape(N_PAD, 2, DH // 32, 2, 16)
           .transpose(1, 0, 2, 4, 3)
           .reshape(2 * N_PAD, DH).astype(jnp.bfloat16))
    orig2d = jnp.pad(is_original_node.astype(f32), (0, npad)).reshape(N_PAD, 1)

    zeros128 = jnp.zeros((N_PAD, DH), f32)
    zeros1 = jnp.zeros((N_PAD,), f32)

    degp = _sc_deg(col_p, ew_p, mn_p, mf_p, zeros1)
    dinv2 = _tc_dinv(degp)
    dn = jnp.asarray(dinv2[0])
    df = jnp.asarray(dinv2[1])

    dnr, dnc, dfr, dfc = _sc_gather_dinv(row_p, col_p, dn, df)
    e2 = (E_PAD // 128, 128)
    normcat = _tc_norms(
        dnr.reshape(e2), dnc.reshape(e2), dfr.reshape(e2), dfc.reshape(e2),
        ew_p.reshape(e2), mn_p.reshape(e2), mf_p.reshape(e2)
    ).reshape(2, E_PAD)
    normn = jnp.asarray(normcat[0])
    normf = jnp.asarray(normcat[1])

    sboth = _sc_agg1(xh, row_p, col_p, normcat, zeros128)
    snh = jnp.asarray(sboth[0])
    sfh = jnp.asarray(sboth[1])
    xh3 = xh.reshape(2, N_PAD, DH)

    w24 = jnp.concatenate(
        [W2n, W2f, jnp.zeros((D, 12), f32)], axis=1)
    g1wh = g1W[:, 0].reshape(2, DH)
    g2wr = g2W[:, 0].reshape(1, D)
    y16, gsc = _tc_dense1(
        xh3, snh, sfh, dn.reshape(N_PAD, 1), df.reshape(N_PAD, 1), orig2d,
        W1n, W1f, w24, g1wh, g2wr,
        b1n.reshape(1, D), b1f.reshape(1, D),
        g1b.reshape(1, 1), g2b.reshape(1, 1))

    ycols = [jnp.asarray(y16[:, i]) for i in range(4)]
    u = _sc_agg2(*ycols, row_p, col_p, normn, normf, zeros1)

    us8 = jnp.stack(
        [u[cc, i] for cc in range(2) for i in range(4)], axis=1)
    out = _tc_final(us8, y16, gsc, b2n.reshape(1, 2), b2f.reshape(1, 2))
    return out[:N]


# speedup vs baseline: 1.6647x; 1.0004x over previous
import functools

import jax
import jax.numpy as jnp
from jax import lax
from jax.experimental import pallas as pl
from jax.experimental.pallas import tpu as pltpu
from jax.experimental.pallas import tpu_sc as plsc

N = 10000
N_PAD = 10240
E = 160000
E_PAD = 163840
D = 256
DQ = 64
DH = 128
NC = 2
NS = 16
EPT = E_PAD // (NC * NS)
EPS = E_PAD // NS
ROWS_PER_SUB = N_PAD // NS
B = 128
CHUNK = 1024


def _deg_body(col_hbm, ew_hbm, mn_hbm, mf_hbm, zeros1_hbm, degp_hbm,
              colb0, colb1, ewb0, ewb1, mnb0, mnb1, mfb0, mfb1,
              msgn0, msgn1, msgf0, msgf1, degn_sh, degf_sh,
              sem_i, sem_s):
    c = lax.axis_index("c")
    s = lax.axis_index("s")
    wid = s * NC + c
    ebase = wid * EPT
    NB = EPT // B
    colbs = (colb0, colb1)
    ewbs = (ewb0, ewb1)
    mnbs = (mnb0, mnb1)
    mfbs = (mfb0, mfb1)
    msgns = (msgn0, msgn1)
    msgfs = (msgf0, msgf1)

    pltpu.sync_copy(zeros1_hbm.at[pl.ds(s * ROWS_PER_SUB, ROWS_PER_SUB)],
                    degn_sh.at[pl.ds(s * ROWS_PER_SUB, ROWS_PER_SUB)])
    pltpu.sync_copy(zeros1_hbm.at[pl.ds(s * ROWS_PER_SUB, ROWS_PER_SUB)],
                    degf_sh.at[pl.ds(s * ROWS_PER_SUB, ROWS_PER_SUB)])
    plsc.subcore_barrier()

    def stage(bset, off):
        pltpu.async_copy(col_hbm.at[pl.ds(off, B)], colbs[bset], sem_i)
        pltpu.async_copy(ew_hbm.at[pl.ds(off, B)], ewbs[bset], sem_i)
        pltpu.async_copy(mn_hbm.at[pl.ds(off, B)], mnbs[bset], sem_i)
        pltpu.async_copy(mf_hbm.at[pl.ds(off, B)], mfbs[bset], sem_i)

    def drain_stage(bset, off):
        pltpu.make_async_copy(
            col_hbm.at[pl.ds(off, B)], colbs[bset], sem_i).wait()
        pltpu.make_async_copy(
            ew_hbm.at[pl.ds(off, B)], ewbs[bset], sem_i).wait()
        pltpu.make_async_copy(
            mn_hbm.at[pl.ds(off, B)], mnbs[bset], sem_i).wait()
        pltpu.make_async_copy(
            mf_hbm.at[pl.ds(off, B)], mfbs[bset], sem_i).wait()

    def drain_scatters(bset):
        pltpu.make_async_copy(
            msgns[bset], degn_sh.at[colbs[bset]], sem_s).wait()
        pltpu.make_async_copy(
            msgfs[bset], degf_sh.at[colbs[bset]], sem_s).wait()

    stage(0, ebase)

    def pair_body(jj, _):
        for par in range(2):
            j = jj * 2 + par
            off = ebase + j * B

            @pl.when(j >= 1)
            def _():
                drain_scatters(1 - par)

            @pl.when(j + 1 < NB)
            def _():
                stage(1 - par, off + B)

            drain_stage(par, off)
            for g in range(B // 16):
                k = g * 16
                w = ewbs[par][pl.ds(k, 16)]
                msgns[par][pl.ds(k, 16)] = w * mnbs[par][pl.ds(k, 16)]
                msgfs[par][pl.ds(k, 16)] = w * mfbs[par][pl.ds(k, 16)]
            pltpu.async_copy(
                msgns[par], degn_sh.at[colbs[par]], sem_s, add=True)
            pltpu.async_copy(
                msgfs[par], degf_sh.at[colbs[par]], sem_s, add=True)
        return 0

    lax.fori_loop(0, NB // 2, pair_body, 0)
    drain_scatters(1)
    plsc.subcore_barrier()
    pltpu.sync_copy(degn_sh.at[pl.ds(s * ROWS_PER_SUB, ROWS_PER_SUB)],
                    degp_hbm.at[c, 0, pl.ds(s * ROWS_PER_SUB, ROWS_PER_SUB)])
    pltpu.sync_copy(degf_sh.at[pl.ds(s * ROWS_PER_SUB, ROWS_PER_SUB)],
                    degp_hbm.at[c, 1, pl.ds(s * ROWS_PER_SUB, ROWS_PER_SUB)])


def _sc_deg(col_p, ew_p, mn_p, mf_p, zeros1):
    mesh = plsc.VectorSubcoreMesh(core_axis_name="c", subcore_axis_name="s")
    f = pl.kernel(
        _deg_body,
        out_type=jax.ShapeDtypeStruct((NC, 2, N_PAD), jnp.float32),
        mesh=mesh,
        scratch_types=(
            [pltpu.VMEM((B,), jnp.int32)] * 2
            + [pltpu.VMEM((B,), jnp.float32)] * 10
            + [pltpu.VMEM_SHARED((N_PAD,), jnp.float32)] * 2
            + [pltpu.SemaphoreType.DMA] * 2
        ),
    )
    return f(col_p, ew_p, mn_p, mf_p, zeros1)


def _dinv_tc_body(degp_ref, dinv_ref):
    deg = degp_ref[0] + degp_ref[1] + 1.0
    dinv_ref[...] = lax.rsqrt(deg)


def _tc_dinv(degp):
    return pl.pallas_call(
        _dinv_tc_body,
        out_shape=jax.ShapeDtypeStruct((2, N_PAD), jnp.float32),
    )(degp)


def _gat_body(row_hbm, col_hbm, dn_hbm, df_hbm,
              dnr_hbm, dnc_hbm, dfr_hbm, dfc_hbm,
              rowb0, rowb1, colb0, colb1,
              ga0, ga1, ga2, ga3, gb0, gb1, gb2, gb3,
              sem_i, sem_g, sem_w):
    c = lax.axis_index("c")
    s = lax.axis_index("s")
    wid = s * NC + c
    ebase = wid * EPT
    NB = EPT // B
    rowbs = (rowb0, rowb1)
    colbs = (colb0, colb1)
    gsets = ((ga0, ga1, ga2, ga3), (gb0, gb1, gb2, gb3))
    outs = (dnr_hbm, dnc_hbm, dfr_hbm, dfc_hbm)

    def gather_all(bset):
        r, cc = rowbs[bset], colbs[bset]
        g = gsets[bset]
        pltpu.async_copy(dn_hbm.at[r], g[0], sem_g)
        pltpu.async_copy(dn_hbm.at[cc], g[1], sem_g)
        pltpu.async_copy(df_hbm.at[r], g[2], sem_g)
        pltpu.async_copy(df_hbm.at[cc], g[3], sem_g)

    def drain_gathers(bset):
        r, cc = rowbs[bset], colbs[bset]
        g = gsets[bset]
        pltpu.make_async_copy(dn_hbm.at[r], g[0], sem_g).wait()
        pltpu.make_async_copy(dn_hbm.at[cc], g[1], sem_g).wait()
        pltpu.make_async_copy(df_hbm.at[r], g[2], sem_g).wait()
        pltpu.make_async_copy(df_hbm.at[cc], g[3], sem_g).wait()

    def drain_writes(bset, off):
        for i in range(4):
            pltpu.make_async_copy(
                gsets[bset][i], outs[i].at[pl.ds(off, B)], sem_w).wait()

    pltpu.sync_copy(row_hbm.at[pl.ds(ebase, B)], rowb0)
    pltpu.sync_copy(col_hbm.at[pl.ds(ebase, B)], colb0)
    gather_all(0)

    def pair_body(jj, _):
        for par in range(2):
            j = jj * 2 + par
            off = ebase + j * B
            nxt = ebase + (j + 1) * B

            @pl.when(j + 1 < NB)
            def _():
                pltpu.async_copy(
                    row_hbm.at[pl.ds(nxt, B)], rowbs[1 - par], sem_i)
                pltpu.async_copy(
                    col_hbm.at[pl.ds(nxt, B)], colbs[1 - par], sem_i)

            drain_gathers(par)
            for i in range(4):
                pltpu.async_copy(
                    gsets[par][i], outs[i].at[pl.ds(off, B)], sem_w)

            @pl.when(j >= 1)
            def _():
                drain_writes(1 - par, off - B)

            @pl.when(j + 1 < NB)
            def _():
                pltpu.make_async_copy(
                    row_hbm.at[pl.ds(nxt, B)], rowbs[1 - par], sem_i).wait()
                pltpu.make_async_copy(
                    col_hbm.at[pl.ds(nxt, B)], colbs[1 - par], sem_i).wait()
                gather_all(1 - par)
        return 0

    lax.fori_loop(0, NB // 2, pair_body, 0)
    drain_writes(1, ebase + (NB - 1) * B)


def _sc_gather_dinv(row_p, col_p, dn, df):
    mesh = plsc.VectorSubcoreMesh(core_axis_name="c", subcore_axis_name="s")
    f = pl.kernel(
        _gat_body,
        out_type=tuple(
            jax.ShapeDtypeStruct((E_PAD,), jnp.float32) for _ in range(4)),
        mesh=mesh,
        scratch_types=(
            [pltpu.VMEM((B,), jnp.int32)] * 4
            + [pltpu.VMEM((B,), jnp.float32)] * 8
            + [pltpu.SemaphoreType.DMA] * 3
        ),
    )
    return f(row_p, col_p, dn, df)


def _norm_tc_body(dnr_ref, dnc_ref, dfr_ref, dfc_ref, ew_ref, mn_ref, mf_ref,
                  normcat_ref):
    w = ew_ref[...]
    normcat_ref[0] = dnr_ref[...] * dnc_ref[...] * (w * mn_ref[...])
    normcat_ref[1] = dfr_ref[...] * dfc_ref[...] * (w * mf_ref[...])


def _tc_norms(dnr, dnc, dfr, dfc, ew2, mn2, mf2):
    return pl.pallas_call(
        _norm_tc_body,
        out_shape=jax.ShapeDtypeStruct((2, E_PAD // 128, 128), jnp.float32),
    )(dnr, dnc, dfr, dfc, ew2, mn2, mf2)


RING = 2
B1 = 128


def _agg1_body(xh_hbm, row_hbm, col_hbm, normcat_hbm, zeros_hbm,
               s_hbm,
               normv, rowb0, rowb1, colb0, colb1, xr0, xr1, acc_sh,
               sem_g, sem_s):
    c = lax.axis_index("c")
    s = lax.axis_index("s")
    ebase = s * EPS
    NB = EPS // B1
    rowbs = (rowb0, rowb1)
    colbs = (colb0, colb1)
    xrs = (xr0, xr1)

    pltpu.sync_copy(normcat_hbm.at[c, pl.ds(ebase, EPS)], normv)

    for h in range(2):
        hoff = h * N_PAD

        def stage(bset, j):
            off = ebase + j * B1
            pltpu.sync_copy(col_hbm.at[pl.ds(off, B1)], colbs[bset])
            pltpu.sync_copy(row_hbm.at[pl.ds(off, B1)], rowbs[bset])
            if h:
                def ibody(i, _):
                    k = i * 16
                    rowbs[bset][pl.ds(k, 16)] = (
                        rowbs[bset][pl.ds(k, 16)] + hoff)
                    return 0
                lax.fori_loop(0, B1 // 16, ibody, 0)
            pltpu.async_copy(xh_hbm.at[rowbs[bset]], xrs[bset], sem_g)

        pltpu.sync_copy(zeros_hbm.at[pl.ds(s * ROWS_PER_SUB, ROWS_PER_SUB)],
                        acc_sh.at[pl.ds(s * ROWS_PER_SUB, ROWS_PER_SUB)])
        plsc.subcore_barrier()
        stage(0, 0)

        def pair_body(jj, _):
            for par in range(2):
                j2 = jj * 2 + par
                xr = xrs[par]
                pltpu.make_async_copy(
                    xh_hbm.at[rowbs[par]], xr, sem_g).wait()

                @pl.when(j2 >= 1)
                def _():
                    pltpu.make_async_copy(
                        xrs[1 - par], acc_sh.at[colbs[1 - par]], sem_s).wait()

                @pl.when(j2 + 1 < NB)
                def _():
                    stage(1 - par, j2 + 1)

                def scale_body(g, _):
                    nng = normv[pl.ds(j2 * B1 + g * 16, 16)]
                    for ke in range(16):
                        e = g * 16 + ke
                        nn = nng[ke]
                        for k in range(DH // 16):
                            xr[e, pl.ds(k * 16, 16)] = (
                                xr[e, pl.ds(k * 16, 16)] * nn)
                    return 0

                lax.fori_loop(0, B1 // 16, scale_body, 0, unroll=2)
                pltpu.async_copy(xr, acc_sh.at[colbs[par]], sem_s,
                                 add=True)
            return 0

        lax.fori_loop(0, NB // 2, pair_body, 0)
        pltpu.make_async_copy(xrs[1], acc_sh.at[colbs[1]], sem_s).wait()
        plsc.subcore_barrier()

        r0 = s * ROWS_PER_SUB
        pltpu.sync_copy(acc_sh.at[pl.ds(r0, ROWS_PER_SUB)],
                        s_hbm.at[c, h, pl.ds(r0, ROWS_PER_SUB)])
        plsc.subcore_barrier()


def _sc_agg1(xhb, row_p, col_p, normcat, zeros128):
    mesh = plsc.VectorSubcoreMesh(core_axis_name="c", subcore_axis_name="s")
    f = pl.kernel(
        _agg1_body,
        out_type=jax.ShapeDtypeStruct((2, 2, N_PAD, DH), jnp.float32),
        mesh=mesh,
        scratch_types=(
            [pltpu.VMEM((EPS,), jnp.float32)]
            + [pltpu.VMEM((B1,), jnp.int32)] * 4
            + [pltpu.VMEM((B1, DH), jnp.float32)] * 2
            + [pltpu.VMEM_SHARED((N_PAD, DH), jnp.float32)]
            + [pltpu.SemaphoreType.DMA, pltpu.SemaphoreType.DMA]
        ),
    )
    return f(xhb, row_p, col_p, normcat, zeros128)


def _dense1_body(x_ref, sn_ref, sf_ref, dn_ref, df_ref, orig_ref,
                 w1n_ref, w1f_ref, w24_ref, g1w_ref, g2w_ref,
                 b1n_ref, b1f_ref, g1b_ref, g2b_ref,
                 y4_ref, gsc_ref):
    dn2 = dn_ref[...] * dn_ref[...]
    df2 = df_ref[...] * df_ref[...]
    orig = orig_ref[...]

    br = x_ref.shape[1]
    accn = jnp.zeros((br, D), jnp.float32)
    accf = jnp.zeros((br, D), jnp.float32)
    g1a = jnp.zeros((br, 1), jnp.float32)
    for h in range(2):
        xh = x_ref[h]
        tnh = sn_ref[h] + dn2 * xh
        tfh = sf_ref[h] + df2 * xh
        w1n_h = w1n_ref[pl.ds(h * DH, DH), :]
        w1f_h = w1f_ref[pl.ds(h * DH, DH), :]
        accn = accn + jnp.dot(tnh, w1n_h, preferred_element_type=jnp.float32)
        accf = accf + jnp.dot(tfh, w1f_h, preferred_element_type=jnp.float32)
        g1a = g1a + jnp.sum(xh * g1w_ref[h:h + 1, :], axis=1, keepdims=True)

    hn1 = accn + b1n_ref[...]
    hf1 = accf + b1f_ref[...]
    g1 = jax.nn.sigmoid(g1a + g1b_ref[0, 0])
    h1 = jax.nn.relu((g1 * hn1 + (1.0 - g1) * hf1) * orig)
    y16 = jnp.dot(h1, w24_ref[...], preferred_element_type=jnp.float32)
    g2 = jax.nn.sigmoid(
        jnp.sum(h1 * g2w_ref[...], axis=1, keepdims=True) + g2b_ref[0, 0])
    y4_ref[...] = y16
    gsc_ref[...] = jnp.concatenate(
        [g2, dn2, df2, orig, jnp.zeros((br, 12), jnp.float32)], axis=1)


def _tc_dense1(xq, snq, sfq, dn2d, df2d, orig2d, w1n, w1f, w24, g1wq, g2wr,
               b1n, b1f, g1b, g2b):
    BR = 512
    grid = (N_PAD // BR,)
    return pl.pallas_call(
        _dense1_body,
        grid=grid,
        in_specs=[
            pl.BlockSpec((2, BR, DH), lambda i: (0, i, 0)),
            pl.BlockSpec((2, BR, DH), lambda i: (0, i, 0)),
            pl.BlockSpec((2, BR, DH), lambda i: (0, i, 0)),
            pl.BlockSpec((BR, 1), lambda i: (i, 0)),
            pl.BlockSpec((BR, 1), lambda i: (i, 0)),
            pl.BlockSpec((BR, 1), lambda i: (i, 0)),
            pl.BlockSpec((D, D), lambda i: (0, 0)),
            pl.BlockSpec((D, D), lambda i: (0, 0)),
            pl.BlockSpec((D, 16), lambda i: (0, 0)),
            pl.BlockSpec((2, DH), lambda i: (0, 0)),
            pl.BlockSpec((1, D), lambda i: (0, 0)),
            pl.BlockSpec((1, D), lambda i: (0, 0)),
            pl.BlockSpec((1, D), lambda i: (0, 0)),
            pl.BlockSpec((1, 1), lambda i: (0, 0)),
            pl.BlockSpec((1, 1), lambda i: (0, 0)),
        ],
        out_specs=[
            pl.BlockSpec((BR, 16), lambda i: (i, 0)),
            pl.BlockSpec((BR, 16), lambda i: (i, 0)),
        ],
        out_shape=(
            jax.ShapeDtypeStruct((N_PAD, 16), jnp.float32),
            jax.ShapeDtypeStruct((N_PAD, 16), jnp.float32),
        ),
    )(xq, snq, sfq, dn2d, df2d, orig2d, w1n, w1f, w24, g1wq, g2wr,
      b1n, b1f, g1b, g2b)


def _agg2_body(y0_hbm, y1_hbm, y2_hbm, y3_hbm, row_hbm, col_hbm,
               normn_hbm, normf_hbm, zeros1_hbm,
               u_hbm,
               normnv, normfv, rowb0, rowb1, colb0, colb1,
               ga0, ga1, ga2, ga3, gb0, gb1, gb2, gb3,
               ma0, ma1, ma2, ma3, mb0, mb1, mb2, mb3,
               a0, a1, a2, a3, sem_i, sem_g, sem_s):
    c = lax.axis_index("c")
    s = lax.axis_index("s")
    wid = s * NC + c
    ebase = wid * EPT
    ys = (y0_hbm, y1_hbm, y2_hbm, y3_hbm)
    rowbs = (rowb0, rowb1)
    colbs = (colb0, colb1)
    gs = ((ga0, ga1, ga2, ga3), (gb0, gb1, gb2, gb3))
    ms = ((ma0, ma1, ma2, ma3), (mb0, mb1, mb2, mb3))
    accs = (a0, a1, a2, a3)

    pltpu.sync_copy(normn_hbm.at[pl.ds(ebase, EPT)], normnv)
    pltpu.sync_copy(normf_hbm.at[pl.ds(ebase, EPT)], normfv)
    for acc in accs:
        pltpu.sync_copy(zeros1_hbm.at[pl.ds(s * ROWS_PER_SUB, ROWS_PER_SUB)],
                        acc.at[pl.ds(s * ROWS_PER_SUB, ROWS_PER_SUB)])
    plsc.subcore_barrier()
    NB = EPT // B

    def gather_all(par):
        for i in range(4):
            pltpu.async_copy(ys[i].at[rowbs[par]], gs[par][i], sem_g)

    def drain_gathers(par):
        for i in range(4):
            pltpu.make_async_copy(
                ys[i].at[rowbs[par]], gs[par][i], sem_g).wait()

    def drain_scatters(par):
        for i in range(4):
            pltpu.make_async_copy(
                ms[par][i], accs[i].at[colbs[par]], sem_s).wait()

    pltpu.sync_copy(row_hbm.at[pl.ds(ebase, B)], rowbs[0])
    pltpu.sync_copy(col_hbm.at[pl.ds(ebase, B)], colbs[0])
    gather_all(0)

    def pair_body(jj, _):
        for par in range(2):
            j = jj * 2 + par
            eb = j * B
            nxt = ebase + (j + 1) * B

            @pl.when(j >= 1)
            def _():
                drain_scatters(1 - par)

            @pl.when(j + 1 < NB)
            def _():
                pltpu.async_copy(
                    row_hbm.at[pl.ds(nxt, B)], rowbs[1 - par], sem_i)
                pltpu.async_copy(
                    col_hbm.at[pl.ds(nxt, B)], colbs[1 - par], sem_i)

            drain_gathers(par)

            def group_body(g, _):
                k = g * 16
                nng = normnv[pl.ds(eb + k, 16)]
                nfg = normfv[pl.ds(eb + k, 16)]
                ms[par][0][pl.ds(k, 16)] = gs[par][0][pl.ds(k, 16)] * nng
                ms[par][1][pl.ds(k, 16)] = gs[par][1][pl.ds(k, 16)] * nng
                ms[par][2][pl.ds(k, 16)] = gs[par][2][pl.ds(k, 16)] * nfg
                ms[par][3][pl.ds(k, 16)] = gs[par][3][pl.ds(k, 16)] * nfg
                return 0

            lax.fori_loop(0, B // 16, group_body, 0)
            for i in range(4):
                pltpu.async_copy(
                    ms[par][i], accs[i].at[colbs[par]], sem_s, add=True)

            @pl.when(j + 1 < NB)
            def _():
                pltpu.make_async_copy(
                    row_hbm.at[pl.ds(nxt, B)], rowbs[1 - par], sem_i).wait()
                pltpu.make_async_copy(
                    col_hbm.at[pl.ds(nxt, B)], colbs[1 - par], sem_i).wait()
                gather_all(1 - par)
        return 0

    lax.fori_loop(0, NB // 2, pair_body, 0)
    drain_scatters(1)
    plsc.subcore_barrier()

    r0 = s * ROWS_PER_SUB
    for i in range(4):
        pltpu.sync_copy(accs[i].at[pl.ds(r0, ROWS_PER_SUB)],
                        u_hbm.at[c, i, pl.ds(r0, ROWS_PER_SUB)])


def _sc_agg2(y0, y1, y2, y3, row_p, col_p, normn, normf, zeros1):
    mesh = plsc.VectorSubcoreMesh(core_axis_name="c", subcore_axis_name="s")
    f = pl.kernel(
        _agg2_body,
        out_type=jax.ShapeDtypeStruct((NC, 4, N_PAD), jnp.float32),
        mesh=mesh,
        scratch_types=(
            [pltpu.VMEM((EPT,), jnp.float32)] * 2
            + [pltpu.VMEM((B,), jnp.int32)] * 4
            + [pltpu.VMEM((B,), jnp.float32)] * 16
            + [pltpu.VMEM_SHARED((N_PAD,), jnp.float32)] * 4
            + [pltpu.SemaphoreType.DMA] * 3
        ),
    )
    return f(y0, y1, y2, y3, row_p, col_p, normn, normf, zeros1)


def _final_body(us_ref, y16_ref, gsc_ref, b2n_ref, b2f_ref, out_ref):
    g2 = gsc_ref[:, 0:1]
    dn2 = gsc_ref[:, 1:2]
    df2 = gsc_ref[:, 2:3]
    orig = gsc_ref[:, 3:4]
    un = us_ref[:, 0:2] + us_ref[:, 4:6]
    uf = us_ref[:, 2:4] + us_ref[:, 6:8]
    hn2 = un + dn2 * y16_ref[:, 0:2] + b2n_ref[...]
    hf2 = uf + df2 * y16_ref[:, 2:4] + b2f_ref[...]
    out_ref[...] = orig * (g2 * hn2 + (1.0 - g2) * hf2)


def _tc_final(us8, y16, gsc, b2n2d, b2f2d):
    return pl.pallas_call(
        _final_body,
        out_shape=jax.ShapeDtypeStruct((N_PAD, 2), jnp.float32),
    )(us8, y16, gsc, b2n2d, b2f2d)


def kernel(x, edge_index, edge_weight, node_node_mask, node_feat_mask,
           is_original_node, W1n, b1n, W1f, b1f, W2n, b2n, W2f, b2f,
           g1W, g1b, g2W, g2b):
    f32 = jnp.float32
    row = edge_index[0]
    col = edge_index[1]
    epad = E_PAD - E
    row_p = jnp.pad(row, (0, epad))
    col_p = jnp.pad(col, (0, epad))
    ew_p = jnp.pad(edge_weight, (0, epad))
    mn_p = jnp.pad(node_node_mask.astype(f32), (0, epad))
    mf_p = jnp.pad(node_feat_mask.astype(f32), (0, epad))

    npad = N_PAD - N
    x_pad = jnp.pad(x, ((0, npad), (0, 0)))
    xh = x_pad.reshape(N_PAD, 2, DH).transpose(1, 0, 2).reshape(2 * N_PAD, DH)
    xhb = (x_pad.reshape(N_PAD, 2, DH // 32, 2, 16)
           .transpose(1, 0, 2, 4, 3)
           .reshape(2 * N_PAD, DH).astype(jnp.bfloat16))
    orig2d = jnp.pad(is_original_node.astype(f32), (0, npad)).reshape(N_PAD, 1)

    zeros128 = jnp.zeros((N_PAD, DH), f32)
    zeros1 = jnp.zeros((N_PAD,), f32)

    degp = _sc_deg(col_p, ew_p, mn_p, mf_p, zeros1)
    dinv2 = _tc_dinv(degp)
    dn = jnp.asarray(dinv2[0])
    df = jnp.asarray(dinv2[1])

    dnr, dnc, dfr, dfc = _sc_gather_dinv(row_p, col_p, dn, df)
    e2 = (E_PAD // 128, 128)
    normcat = _tc_norms(
        dnr.reshape(e2), dnc.reshape(e2), dfr.reshape(e2), dfc.reshape(e2),
        ew_p.reshape(e2), mn_p.reshape(e2), mf_p.reshape(e2)
    ).reshape(2, E_PAD)
    normn = jnp.asarray(normcat[0])
    normf = jnp.asarray(normcat[1])

    sboth = _sc_agg1(xh, row_p, col_p, normcat, zeros128)
    snh = jnp.asarray(sboth[0])
    sfh = jnp.asarray(sboth[1])
    xh3 = xh.reshape(2, N_PAD, DH)

    w24 = jnp.concatenate(
        [W2n, W2f, jnp.zeros((D, 12), f32)], axis=1)
    g1wh = g1W[:, 0].reshape(2, DH)
    g2wr = g2W[:, 0].reshape(1, D)
    y16, gsc = _tc_dense1(
        xh3, snh, sfh, dn.reshape(N_PAD, 1), df.reshape(N_PAD, 1), orig2d,
        W1n, W1f, w24, g1wh, g2wr,
        b1n.reshape(1, D), b1f.reshape(1, D),
        g1b.reshape(1, 1), g2b.reshape(1, 1))

    ycols = [jnp.asarray(y16[:, i]) for i in range(4)]
    u = _sc_agg2(*ycols, row_p, col_p, normn, normf, zeros1)

    us8 = jnp.stack(
        [u[cc, i] for cc in range(2) for i in range(4)], axis=1)
    out = _tc_final(us8, y16, gsc, b2n.reshape(1, 2), b2f.reshape(1, 2))
    return out[:N]
